# Initial kernel scaffold; baseline (speedup 1.0000x reference)
import jax
import jax.numpy as jnp
from jax.experimental import pallas as pl


def kernel(surface_features, knn_indices, point_mask, W_in, b_in, W_conv, b_conv, gamma, beta, W_h1, b_h1, W_h2, b_h2):
    raise NotImplementedError("write your pallas kernel here")


# speedup vs baseline: 1.5684x; 1.5684x over previous
import jax
import jax.numpy as jnp
from jax import lax
from jax.experimental import pallas as pl
from jax.experimental.pallas import tpu as pltpu
from jax.experimental.pallas import tpu_sc as plsc

N = 10000
K = 32
D = 11
H = 128
L = 3

NW = 32
NPW = 320
NPAD = NW * NPW
GN = 4
NIDX = GN * K
NG = NPW // GN
DP = 16
ROWS_BLK = 512
EPS = 1e-5

_SC_MESH = plsc.VectorSubcoreMesh(
    core_axis_name="c", subcore_axis_name="s", num_cores=2, num_subcores=16
)


def _sc_gather_mean(x_hbm, idx_hbm, out_hbm, idx_v, rows0, rows1, agg_v, sem0, sem1):
    wid = lax.axis_index("s") * 2 + lax.axis_index("c")
    pltpu.sync_copy(idx_hbm.at[wid], idx_v)

    def accum(rows, g):
        for b in range(GN):
            nl = g * GN + b

            def kbody(k, accs):
                return tuple(
                    accs[h] + rows[b * K + k, pl.ds(h * 16, 16)] for h in range(8)
                )

            accs = lax.fori_loop(
                0, K, kbody, tuple(jnp.zeros((16,), jnp.float32) for _ in range(8))
            )
            for h in range(8):
                agg_v[nl, pl.ds(h * 16, 16)] = accs[h] * (1.0 / K)

    def body(i, carry):
        g0 = 2 * i
        g1 = g0 + 1
        cp0 = pltpu.async_copy(x_hbm.at[idx_v.at[g0]], rows0, sem0)
        cp1 = pltpu.async_copy(x_hbm.at[idx_v.at[g1]], rows1, sem1)
        cp0.wait()
        accum(rows0, g0)
        cp1.wait()
        accum(rows1, g1)
        return carry

    lax.fori_loop(0, NG // 2, body, 0)
    pltpu.sync_copy(agg_v, out_hbm.at[pl.ds(wid * NPW, NPW)])


_sc_gather = pl.kernel(
    _sc_gather_mean,
    out_type=jax.ShapeDtypeStruct((NPAD, H), jnp.float32),
    mesh=_SC_MESH,
    scratch_types=[
        pltpu.VMEM((NG, NIDX), jnp.int32),
        pltpu.VMEM((NIDX, H), jnp.float32),
        pltpu.VMEM((NIDX, H), jnp.float32),
        pltpu.VMEM((NPW, H), jnp.float32),
        pltpu.SemaphoreType.DMA,
        pltpu.SemaphoreType.DMA,
    ],
)


def _in_proj_body(f_ref, w_ref, b_ref, o_ref):
    o_ref[...] = (
        jnp.dot(f_ref[...], w_ref[...], preferred_element_type=jnp.float32)
        + b_ref[...]
    )


def _in_proj(feat, w, b):
    return pl.pallas_call(
        _in_proj_body,
        grid=(NPAD // ROWS_BLK,),
        in_specs=[
            pl.BlockSpec((ROWS_BLK, DP), lambda i: (i, 0)),
            pl.BlockSpec((DP, H), lambda i: (0, 0)),
            pl.BlockSpec((1, H), lambda i: (0, 0)),
        ],
        out_specs=pl.BlockSpec((ROWS_BLK, H), lambda i: (i, 0)),
        out_shape=jax.ShapeDtypeStruct((NPAD, H), jnp.float32),
    )(feat, w, b)


def _layer_update(x, agg, w, b, g, bt):
    y = x + jnp.dot(agg, w, preferred_element_type=jnp.float32) + b
    mu = jnp.mean(y, axis=-1, keepdims=True)
    var = jnp.mean((y - mu) ** 2, axis=-1, keepdims=True)
    y = (y - mu) * lax.rsqrt(var + EPS) * g + bt
    return jnp.maximum(y, 0.0)


def _layer_body(x_ref, a_ref, w_ref, b_ref, g_ref, bt_ref, o_ref):
    o_ref[...] = _layer_update(
        x_ref[...], a_ref[...], w_ref[...], b_ref[...], g_ref[...], bt_ref[...]
    )


def _layer(x, agg, w, b, g, bt):
    return pl.pallas_call(
        _layer_body,
        grid=(NPAD // ROWS_BLK,),
        in_specs=[
            pl.BlockSpec((ROWS_BLK, H), lambda i: (i, 0)),
            pl.BlockSpec((ROWS_BLK, H), lambda i: (i, 0)),
            pl.BlockSpec((H, H), lambda i: (0, 0)),
            pl.BlockSpec((1, H), lambda i: (0, 0)),
            pl.BlockSpec((1, H), lambda i: (0, 0)),
            pl.BlockSpec((1, H), lambda i: (0, 0)),
        ],
        out_specs=pl.BlockSpec((ROWS_BLK, H), lambda i: (i, 0)),
        out_shape=jax.ShapeDtypeStruct((NPAD, H), jnp.float32),
    )(x, agg, w, b, g, bt)


def _final_body(
    x_ref, a_ref, w_ref, b_ref, g_ref, bt_ref, wh1_ref, bh1_ref, wh2_ref, bh2_ref,
    m_ref, o_ref,
):
    y = _layer_update(
        x_ref[...], a_ref[...], w_ref[...], b_ref[...], g_ref[...], bt_ref[...]
    )
    h = jnp.maximum(
        jnp.dot(y, wh1_ref[...], preferred_element_type=jnp.float32) + bh1_ref[...],
        0.0,
    )
    logit = jnp.sum(h * wh2_ref[...], axis=-1) + bh2_ref[0, 0]
    o_ref[...] = jax.nn.sigmoid(logit) * m_ref[...]


def _final(x, agg, w, b, g, bt, wh1, bh1, wh2, bh2, mask):
    return pl.pallas_call(
        _final_body,
        grid=(NPAD // ROWS_BLK,),
        in_specs=[
            pl.BlockSpec((ROWS_BLK, H), lambda i: (i, 0)),
            pl.BlockSpec((ROWS_BLK, H), lambda i: (i, 0)),
            pl.BlockSpec((H, H), lambda i: (0, 0)),
            pl.BlockSpec((1, H), lambda i: (0, 0)),
            pl.BlockSpec((1, H), lambda i: (0, 0)),
            pl.BlockSpec((1, H), lambda i: (0, 0)),
            pl.BlockSpec((H, H // 2), lambda i: (0, 0)),
            pl.BlockSpec((1, H // 2), lambda i: (0, 0)),
            pl.BlockSpec((1, H // 2), lambda i: (0, 0)),
            pl.BlockSpec((1, 1), lambda i: (0, 0)),
            pl.BlockSpec((ROWS_BLK,), lambda i: (i,)),
        ],
        out_specs=pl.BlockSpec((ROWS_BLK,), lambda i: (i,)),
        out_shape=jax.ShapeDtypeStruct((NPAD,), jnp.float32),
    )(x, agg, w, b, g, bt, wh1, bh1, wh2, bh2, mask)


def kernel(surface_features, knn_indices, point_mask, W_in, b_in, W_conv, b_conv,
           gamma, beta, W_h1, b_h1, W_h2, b_h2):
    feat = jnp.pad(surface_features[0], ((0, NPAD - N), (0, DP - D)))
    w_in = jnp.pad(W_in, ((0, DP - D), (0, 0)))
    idx = jnp.pad(
        knn_indices[0].astype(jnp.int32).reshape(-1), (0, (NPAD - N) * K)
    ).reshape(NW, NG, NIDX)
    mask = jnp.pad(point_mask[0], (0, NPAD - N))

    x = _in_proj(feat, w_in, b_in.reshape(1, H))
    for l in range(L - 1):
        agg = _sc_gather(x, idx)
        x = _layer(
            x, agg, W_conv[l], b_conv[l].reshape(1, H),
            gamma[l].reshape(1, H), beta[l].reshape(1, H),
        )
    agg = _sc_gather(x, idx)
    probs = _final(
        x, agg, W_conv[L - 1], b_conv[L - 1].reshape(1, H),
        gamma[L - 1].reshape(1, H), beta[L - 1].reshape(1, H),
        W_h1, b_h1.reshape(1, H // 2), W_h2.reshape(1, H // 2),
        b_h2.reshape(1, 1), mask,
    )
    return probs[:N][None, :]
